# Initial kernel scaffold; baseline (speedup 1.0000x reference)
import jax
import jax.numpy as jnp
from jax.experimental import pallas as pl


def kernel(x, mask, edge_index, W, b, alpha):
    raise NotImplementedError("write your pallas kernel here")


# speedup vs baseline: 38.5957x; 38.5957x over previous
import functools

import jax
import jax.numpy as jnp
from jax import lax
from jax.experimental import pallas as pl
from jax.experimental.pallas import tpu as pltpu
from jax.experimental.pallas import tpu_sc as plsc

N = 10000
D = 128
E = 320000
K = 5

NC = 2
NS = 16
NW = NC * NS
NP = 10240
SL = NP // NS
EP = 327680
EW_ROWS = 80
EPW = EW_ROWS * 128

_mesh = functools.partial(
    plsc.VectorSubcoreMesh, core_axis_name="c", subcore_axis_name="s",
    num_cores=NC, num_subcores=NS)


def _wid():
    cid = lax.axis_index("c")
    sid = lax.axis_index("s")
    return cid, sid, sid * NC + cid


def _zero_ref(ref, n):
    for g in range(n // 16):
        ref[pl.ds(g * 16, 16)] = jnp.zeros((16,), ref.dtype)


def _tc_hn_body(x_ref, w_ref, b_ref, m_ref, hn_ref, f_ref):
    h = jnp.tanh(
        lax.dot_general(x_ref[...], w_ref[...], (((1,), (1,)), ((), ())),
                        preferred_element_type=jnp.float32) + b_ref[...])
    nrm = jnp.sqrt(jnp.sum(h * h, axis=1, keepdims=True))
    hn_ref[...] = h / jnp.maximum(nrm, 1e-8)
    f_ref[...] = jnp.maximum(m_ref[...], 0.0)


def _tc_hn(x_pad, W, b2, mask2):
    return pl.pallas_call(
        _tc_hn_body,
        grid=(20,),
        in_specs=[
            pl.BlockSpec((512, D), lambda i: (i, 0)),
            pl.BlockSpec((D, D), lambda i: (0, 0)),
            pl.BlockSpec((1, D), lambda i: (0, 0)),
            pl.BlockSpec((EW_ROWS, 128), lambda i: (0, 0)),
        ],
        out_specs=[
            pl.BlockSpec((512, D), lambda i: (i, 0)),
            pl.BlockSpec((EW_ROWS, 128), lambda i: (0, 0)),
        ],
        out_shape=[
            jax.ShapeDtypeStruct((NP, D), jnp.float32),
            jax.ShapeDtypeStruct((EW_ROWS, 128), jnp.float32),
        ],
    )(x_pad, W, b2, mask2)


def _tc_dinv_body(p_ref, o_ref):
    o_ref[...] = lax.rsqrt(p_ref[0] + p_ref[1] + 1.0)


def _tc_dinv(degp2):
    return pl.pallas_call(
        _tc_dinv_body,
        out_shape=jax.ShapeDtypeStruct((EW_ROWS, 128), jnp.float32),
    )(degp2)


def _tc_blend_body(p_ref, o_ref, d_ref, f_ref, a_ref, out_ref):
    a = a_ref[...]
    d = d_ref[...]
    out_ref[...] = ((d * (p_ref[0] + p_ref[1]) + d * d * o_ref[...])
                    * (1.0 - a) + a * f_ref[...])


def _tc_blend(p2, out2, dinv2, fill2, alpha2):
    return pl.pallas_call(
        _tc_blend_body,
        out_shape=jax.ShapeDtypeStruct((EW_ROWS, 128), jnp.float32),
    )(p2, out2, dinv2, fill2, alpha2)


def _sc_edges_body(hn, src3, dst3, ew3, degp,
                   srcv, dstv, rs0, rd0, rs1, rd1, ewall, zbuf,
                   acc, sem):
    cid, sid, wid = _wid()
    _zero_ref(zbuf, SL)
    pltpu.sync_copy(zbuf, acc.at[pl.ds(sid * SL, SL)])
    pltpu.sync_copy(src3.at[wid], srcv)
    pltpu.sync_copy(dst3.at[wid], dstv)
    plsc.subcore_barrier()

    ebase = wid * EPW
    iota16 = lax.iota(jnp.int32, 16)
    perms = [(iota16 + s) % 16 for s in (8, 4, 2, 1)]
    lanemasks = [iota16 == u for u in range(16)]
    zero16 = jnp.zeros((16,), jnp.float32)

    def issue(c, rs, rd):
        pltpu.async_copy(hn.at[srcv.at[c]], rs, sem)
        pltpu.async_copy(hn.at[dstv.at[c]], rd, sem)

    def wait2(rs, rd, c):
        pltpu.make_async_copy(hn.at[srcv.at[c]], rs, sem).wait()
        pltpu.make_async_copy(hn.at[dstv.at[c]], rd, sem).wait()

    def compute(c, rs, rd):
        def group(g, carry2):
            res = zero16
            for u in range(16):
                e = g * 16 + u
                acc16 = rs[e, pl.ds(0, 16)] * rd[e, pl.ds(0, 16)]
                for j in range(1, 8):
                    sl = pl.ds(j * 16, 16)
                    acc16 = acc16 + rs[e, sl] * rd[e, sl]
                for p in perms:
                    acc16 = acc16 + jnp.take(acc16, p)
                res = jnp.where(lanemasks[u], acc16, res)
            ewall[c, pl.ds(g * 16, 16)] = jnp.maximum(res, 0.0)
            return carry2

        lax.fori_loop(0, 8, group, 0)
        pltpu.sync_copy(ewall.at[c], acc.at[dstv.at[c]], add=True)

    nreal = jnp.maximum(jnp.minimum((E - ebase) // 128, EW_ROWS), 0)
    issue(0, rs0, rd0)

    def pair(p, carry):
        c0 = 2 * p
        c1 = c0 + 1
        issue(c1, rs1, rd1)
        wait2(rs0, rd0, c0)
        compute(c0, rs0, rd0)

        @pl.when(c0 + 2 < nreal)
        def _():
            issue(c0 + 2, rs0, rd0)

        wait2(rs1, rd1, c1)
        compute(c1, rs1, rd1)
        return carry

    lax.fori_loop(0, nreal // 2, pair, 0)

    def padrow(c, carry):
        for g in range(8):
            ewall[c, pl.ds(g * 16, 16)] = zero16
        return carry

    lax.fori_loop(nreal, EW_ROWS, padrow, 0)
    pltpu.sync_copy(ewall, ew3.at[wid])
    plsc.subcore_barrier()
    pltpu.sync_copy(acc.at[pl.ds(sid * SL, SL)],
                    degp.at[cid, pl.ds(sid * SL, SL)])


def _sc_edges(hn, src3, dst3):
    kern = pl.kernel(
        _sc_edges_body,
        out_type=[
            jax.ShapeDtypeStruct((NW, EW_ROWS, 128), jnp.float32),
            jax.ShapeDtypeStruct((NC, NP), jnp.float32),
        ],
        mesh=_mesh(),
        scratch_types=[
            pltpu.VMEM((EW_ROWS, 128), jnp.int32),
            pltpu.VMEM((EW_ROWS, 128), jnp.int32),
            pltpu.VMEM((128, D), jnp.float32),
            pltpu.VMEM((128, D), jnp.float32),
            pltpu.VMEM((128, D), jnp.float32),
            pltpu.VMEM((128, D), jnp.float32),
            pltpu.VMEM((EW_ROWS, 128), jnp.float32),
            pltpu.VMEM((SL,), jnp.float32),
            pltpu.VMEM_SHARED((NP,), jnp.float32),
            pltpu.SemaphoreType.DMA,
        ],
    )
    return kern(hn, src3, dst3)


def _sc_iter_body(first, src3, dst3, norm3, fill_h, dinv_h, alpha_h,
                  pprev, opp, pout, oprev_out,
                  fv, p0v, p1v, ov, dv, blend, alphav, zbuf,
                  ob, ob2, srcv, dstv, normv, msgv, otab, acc, sem, sem2):
    cid, sid, wid = _wid()
    off = sid * SL
    cpa = pltpu.async_copy(src3.at[wid], srcv, sem2)
    cpb = pltpu.async_copy(dst3.at[wid], dstv, sem2)
    cpc = pltpu.async_copy(norm3.at[wid], normv, sem2)
    cpf = pltpu.async_copy(fill_h.at[pl.ds(off, SL)], fv, sem)
    cpd = pltpu.async_copy(dinv_h.at[pl.ds(off, SL)], dv, sem)
    if first:
        cpf.wait()
        cpd.wait()
        for g in range(SL // 16):
            sl = pl.ds(g * 16, 16)
            blend[sl] = dv[sl] * fv[sl]
            ov[sl] = fv[sl]
    else:
        cp0 = pltpu.async_copy(pprev.at[0, pl.ds(off, SL)], p0v, sem)
        cp1 = pltpu.async_copy(pprev.at[1, pl.ds(off, SL)], p1v, sem)
        cp2 = pltpu.async_copy(opp.at[pl.ds(off, SL)], ov, sem)
        cp4 = pltpu.async_copy(alpha_h, alphav, sem)
        cpf.wait()
        cpd.wait()
        cp0.wait()
        cp1.wait()
        cp2.wait()
        cp4.wait()
        a16 = alphav[...]
        for g in range(SL // 16):
            sl = pl.ds(g * 16, 16)
            d16 = dv[sl]
            o16 = ((d16 * (p0v[sl] + p1v[sl]) + d16 * d16 * ov[sl])
                   * (1.0 - a16) + a16 * fv[sl])
            blend[sl] = d16 * o16
            ov[sl] = o16
    pltpu.sync_copy(blend, otab.at[pl.ds(off, SL)])

    @pl.when(cid == 0)
    def _():
        pltpu.sync_copy(ov, oprev_out.at[pl.ds(off, SL)])

    _zero_ref(zbuf, SL)
    pltpu.sync_copy(zbuf, acc.at[pl.ds(off, SL)])
    cpa.wait()
    cpb.wait()
    cpc.wait()
    plsc.subcore_barrier()

    def issue(r, b):
        pltpu.async_copy(otab.at[srcv.at[r]], b, sem)

    def wait1(r, b):
        pltpu.make_async_copy(otab.at[srcv.at[r]], b, sem).wait()

    def compute(r, b):
        for g in range(8):
            sl = pl.ds(g * 16, 16)
            msgv[r, sl] = normv[r, sl] * b[sl]
        pltpu.async_copy(msgv.at[r], acc.at[dstv.at[r]], sem2, add=True)

    issue(0, ob)

    def pair(p, carry):
        r0 = 2 * p
        r1 = r0 + 1
        issue(r1, ob2)
        wait1(r0, ob)
        compute(r0, ob)

        @pl.when(r0 + 2 < EW_ROWS)
        def _():
            issue(r0 + 2, ob)

        wait1(r1, ob2)
        compute(r1, ob2)
        return carry

    lax.fori_loop(0, EW_ROWS // 2, pair, 0)
    pltpu.make_async_copy(norm3.at[wid], msgv, sem2).wait()
    plsc.subcore_barrier()
    pltpu.sync_copy(acc.at[pl.ds(off, SL)], pout.at[cid, pl.ds(off, SL)])


def _sc_iter(first, src3, dst3, norm3, fill_p, dinv_p, alpha16, pprev, opp):
    kern = pl.kernel(
        functools.partial(_sc_iter_body, first),
        out_type=[
            jax.ShapeDtypeStruct((NC, NP), jnp.float32),
            jax.ShapeDtypeStruct((NP,), jnp.float32),
        ],
        mesh=_mesh(),
        scratch_types=[
            pltpu.VMEM((SL,), jnp.float32),
            pltpu.VMEM((SL,), jnp.float32),
            pltpu.VMEM((SL,), jnp.float32),
            pltpu.VMEM((SL,), jnp.float32),
            pltpu.VMEM((SL,), jnp.float32),
            pltpu.VMEM((SL,), jnp.float32),
            pltpu.VMEM((16,), jnp.float32),
            pltpu.VMEM((SL,), jnp.float32),
            pltpu.VMEM((128,), jnp.float32),
            pltpu.VMEM((128,), jnp.float32),
            pltpu.VMEM((EW_ROWS, 128), jnp.int32),
            pltpu.VMEM((EW_ROWS, 128), jnp.int32),
            pltpu.VMEM((EW_ROWS, 128), jnp.float32),
            pltpu.VMEM((EW_ROWS, 128), jnp.float32),
            pltpu.VMEM_SHARED((NP,), jnp.float32),
            pltpu.VMEM_SHARED((NP,), jnp.float32),
            pltpu.SemaphoreType.DMA,
            pltpu.SemaphoreType.DMA,
        ],
    )
    return kern(src3, dst3, norm3, fill_p, dinv_p, alpha16, pprev, opp)


def kernel(x, mask, edge_index, W, b, alpha):
    src = edge_index[0]
    dst = edge_index[1]
    src3 = jnp.pad(src, (0, EP - E)).reshape(NW, EW_ROWS, 128)
    dst3 = jnp.pad(dst, (0, EP - E)).reshape(NW, EW_ROWS, 128)
    b2 = b.reshape(1, D)
    x_pad = jnp.pad(x, ((0, NP - N), (0, 0)))
    mask2 = jnp.pad(mask[:, 0], (0, NP - N)).reshape(EW_ROWS, 128)
    alpha16 = jnp.broadcast_to(alpha, (16,))
    alpha2 = jnp.broadcast_to(alpha, (1, 128))

    hn, fill2 = _tc_hn(x_pad, W, b2, mask2)
    fill_p = fill2.reshape(NP)
    ew3, degp = _sc_edges(hn, src3, dst3)
    dinv2 = _tc_dinv(degp.reshape(NC, EW_ROWS, 128))
    dinv_p = dinv2.reshape(NP)

    pprev = jnp.zeros((NC, NP), jnp.float32)
    opp = fill_p
    for t in range(K):
        pprev, opp = _sc_iter(t == 0, src3, dst3, ew3, fill_p, dinv_p,
                              alpha16, pprev, opp)

    out5 = _tc_blend(pprev.reshape(NC, EW_ROWS, 128),
                     opp.reshape(EW_ROWS, 128),
                     dinv_p.reshape(EW_ROWS, 128),
                     fill2, alpha2)
    out = out5.reshape(NP)[:N].reshape(N, 1)
    ew = ew3.reshape(EP)[:E]
    return (out, ew)


# speedup vs baseline: 39.0748x; 1.0124x over previous
import functools

import jax
import jax.numpy as jnp
from jax import lax
from jax.experimental import pallas as pl
from jax.experimental.pallas import tpu as pltpu
from jax.experimental.pallas import tpu_sc as plsc

N = 10000
D = 128
E = 320000
K = 5

NC = 2
NS = 16
NW = NC * NS
NP = 10240
SL = NP // NS
EP = 327680
EW_ROWS = 80
EPW = EW_ROWS * 128

_mesh = functools.partial(
    plsc.VectorSubcoreMesh, core_axis_name="c", subcore_axis_name="s",
    num_cores=NC, num_subcores=NS)


def _wid():
    cid = lax.axis_index("c")
    sid = lax.axis_index("s")
    return cid, sid, sid * NC + cid


def _zero_ref(ref, n):
    for g in range(n // 16):
        ref[pl.ds(g * 16, 16)] = jnp.zeros((16,), ref.dtype)


def _tc_hn_body(x_ref, w_ref, b_ref, m_ref, hn_ref, f_ref):
    h = jnp.tanh(
        lax.dot_general(x_ref[...], w_ref[...], (((1,), (1,)), ((), ())),
                        preferred_element_type=jnp.float32) + b_ref[...])
    nrm = jnp.sqrt(jnp.sum(h * h, axis=1, keepdims=True))
    hn_ref[...] = h / jnp.maximum(nrm, 1e-8)
    f_ref[...] = jnp.maximum(m_ref[...], 0.0)


def _tc_hn(x_pad, W, b2, mask2):
    return pl.pallas_call(
        _tc_hn_body,
        grid=(20,),
        in_specs=[
            pl.BlockSpec((512, D), lambda i: (i, 0)),
            pl.BlockSpec((D, D), lambda i: (0, 0)),
            pl.BlockSpec((1, D), lambda i: (0, 0)),
            pl.BlockSpec((EW_ROWS, 128), lambda i: (0, 0)),
        ],
        out_specs=[
            pl.BlockSpec((512, D), lambda i: (i, 0)),
            pl.BlockSpec((EW_ROWS, 128), lambda i: (0, 0)),
        ],
        out_shape=[
            jax.ShapeDtypeStruct((NP, D), jnp.float32),
            jax.ShapeDtypeStruct((EW_ROWS, 128), jnp.float32),
        ],
    )(x_pad, W, b2, mask2)


def _tc_dinv_body(p_ref, o_ref):
    o_ref[...] = lax.rsqrt(p_ref[0] + p_ref[1] + 1.0)


def _tc_dinv(degp2):
    return pl.pallas_call(
        _tc_dinv_body,
        out_shape=jax.ShapeDtypeStruct((EW_ROWS, 128), jnp.float32),
    )(degp2)


def _tc_blend_body(p_ref, o_ref, d_ref, f_ref, a_ref, out_ref):
    a = a_ref[...]
    d = d_ref[...]
    out_ref[...] = ((d * (p_ref[0] + p_ref[1]) + d * d * o_ref[...])
                    * (1.0 - a) + a * f_ref[...])


def _tc_blend(p2, out2, dinv2, fill2, alpha2):
    return pl.pallas_call(
        _tc_blend_body,
        out_shape=jax.ShapeDtypeStruct((EW_ROWS, 128), jnp.float32),
    )(p2, out2, dinv2, fill2, alpha2)


def _sc_edges_body(hn, src3, dst3, ew3, degp,
                   srcv, dstv, rs0, rd0, rs1, rd1, ewall, zbuf,
                   acc, sem, sem2):
    cid, sid, wid = _wid()
    _zero_ref(zbuf, SL)
    pltpu.sync_copy(zbuf, acc.at[pl.ds(sid * SL, SL)])
    pltpu.sync_copy(src3.at[wid], srcv)
    pltpu.sync_copy(dst3.at[wid], dstv)
    plsc.subcore_barrier()

    ebase = wid * EPW
    iota16 = lax.iota(jnp.int32, 16)
    perms = [(iota16 + s) % 16 for s in (8, 4, 2, 1)]
    lanemasks = [iota16 == u for u in range(16)]
    zero16 = jnp.zeros((16,), jnp.float32)

    def issue(c, rs, rd):
        pltpu.async_copy(hn.at[srcv.at[c]], rs, sem)
        pltpu.async_copy(hn.at[dstv.at[c]], rd, sem)

    def wait2(rs, rd, c):
        pltpu.make_async_copy(hn.at[srcv.at[c]], rs, sem).wait()
        pltpu.make_async_copy(hn.at[dstv.at[c]], rd, sem).wait()

    def compute(c, rs, rd):
        def group(g, carry2):
            res = zero16
            for u in range(16):
                e = g * 16 + u
                acc16 = rs[e, pl.ds(0, 16)] * rd[e, pl.ds(0, 16)]
                for j in range(1, 8):
                    sl = pl.ds(j * 16, 16)
                    acc16 = acc16 + rs[e, sl] * rd[e, sl]
                for p in perms:
                    acc16 = acc16 + jnp.take(acc16, p)
                res = jnp.where(lanemasks[u], acc16, res)
            ewall[c, pl.ds(g * 16, 16)] = jnp.maximum(res, 0.0)
            return carry2

        lax.fori_loop(0, 8, group, 0)
        pltpu.async_copy(ewall.at[c], acc.at[dstv.at[c]], sem2, add=True)

    nreal = jnp.maximum(jnp.minimum((E - ebase) // 128, EW_ROWS), 0)
    issue(0, rs0, rd0)

    def pair(p, carry):
        c0 = 2 * p
        c1 = c0 + 1
        issue(c1, rs1, rd1)
        wait2(rs0, rd0, c0)
        compute(c0, rs0, rd0)

        @pl.when(c0 + 2 < nreal)
        def _():
            issue(c0 + 2, rs0, rd0)

        wait2(rs1, rd1, c1)
        compute(c1, rs1, rd1)
        return carry

    lax.fori_loop(0, nreal // 2, pair, 0)

    def padrow(c, carry):
        for g in range(8):
            ewall[c, pl.ds(g * 16, 16)] = zero16
        return carry

    lax.fori_loop(nreal, EW_ROWS, padrow, 0)
    pltpu.sync_copy(ewall, ew3.at[wid])

    def drain(c, carry):
        pltpu.make_async_copy(ewall.at[0], acc.at[dstv.at[0]], sem2).wait()
        return carry

    lax.fori_loop(0, nreal, drain, 0)
    plsc.subcore_barrier()
    pltpu.sync_copy(acc.at[pl.ds(sid * SL, SL)],
                    degp.at[cid, pl.ds(sid * SL, SL)])


def _sc_edges(hn, src3, dst3):
    kern = pl.kernel(
        _sc_edges_body,
        out_type=[
            jax.ShapeDtypeStruct((NW, EW_ROWS, 128), jnp.float32),
            jax.ShapeDtypeStruct((NC, NP), jnp.float32),
        ],
        mesh=_mesh(),
        scratch_types=[
            pltpu.VMEM((EW_ROWS, 128), jnp.int32),
            pltpu.VMEM((EW_ROWS, 128), jnp.int32),
            pltpu.VMEM((128, D), jnp.float32),
            pltpu.VMEM((128, D), jnp.float32),
            pltpu.VMEM((128, D), jnp.float32),
            pltpu.VMEM((128, D), jnp.float32),
            pltpu.VMEM((EW_ROWS, 128), jnp.float32),
            pltpu.VMEM((SL,), jnp.float32),
            pltpu.VMEM_SHARED((NP,), jnp.float32),
            pltpu.SemaphoreType.DMA,
            pltpu.SemaphoreType.DMA,
        ],
    )
    return kern(hn, src3, dst3)


def _sc_iter_body(first, src3, dst3, norm3, fill_h, dinv_h, alpha_h,
                  pprev, opp, pout, oprev_out,
                  fv, p0v, p1v, ov, dv, blend, alphav, zbuf,
                  ob, ob2, srcv, dstv, normv, msgv, otab, acc, sem, sem2):
    cid, sid, wid = _wid()
    off = sid * SL
    cpa = pltpu.async_copy(src3.at[wid], srcv, sem2)
    cpb = pltpu.async_copy(dst3.at[wid], dstv, sem2)
    cpc = pltpu.async_copy(norm3.at[wid], normv, sem2)
    cpf = pltpu.async_copy(fill_h.at[pl.ds(off, SL)], fv, sem)
    cpd = pltpu.async_copy(dinv_h.at[pl.ds(off, SL)], dv, sem)
    if first:
        cpf.wait()
        cpd.wait()
        for g in range(SL // 16):
            sl = pl.ds(g * 16, 16)
            blend[sl] = dv[sl] * fv[sl]
            ov[sl] = fv[sl]
    else:
        cp0 = pltpu.async_copy(pprev.at[0, pl.ds(off, SL)], p0v, sem)
        cp1 = pltpu.async_copy(pprev.at[1, pl.ds(off, SL)], p1v, sem)
        cp2 = pltpu.async_copy(opp.at[pl.ds(off, SL)], ov, sem)
        cp4 = pltpu.async_copy(alpha_h, alphav, sem)
        cpf.wait()
        cpd.wait()
        cp0.wait()
        cp1.wait()
        cp2.wait()
        cp4.wait()
        a16 = alphav[...]
        for g in range(SL // 16):
            sl = pl.ds(g * 16, 16)
            d16 = dv[sl]
            o16 = ((d16 * (p0v[sl] + p1v[sl]) + d16 * d16 * ov[sl])
                   * (1.0 - a16) + a16 * fv[sl])
            blend[sl] = d16 * o16
            ov[sl] = o16
    pltpu.sync_copy(blend, otab.at[pl.ds(off, SL)])

    @pl.when(cid == 0)
    def _():
        pltpu.sync_copy(ov, oprev_out.at[pl.ds(off, SL)])

    _zero_ref(zbuf, SL)
    pltpu.sync_copy(zbuf, acc.at[pl.ds(off, SL)])
    cpa.wait()
    cpb.wait()
    cpc.wait()
    plsc.subcore_barrier()

    def issue(r, b):
        pltpu.async_copy(otab.at[srcv.at[r]], b, sem)

    def wait1(r, b):
        pltpu.make_async_copy(otab.at[srcv.at[r]], b, sem).wait()

    def compute(r, b):
        for g in range(8):
            sl = pl.ds(g * 16, 16)
            msgv[r, sl] = normv[r, sl] * b[sl]
        pltpu.async_copy(msgv.at[r], acc.at[dstv.at[r]], sem2, add=True)

    issue(0, ob)

    def pair(p, carry):
        r0 = 2 * p
        r1 = r0 + 1
        issue(r1, ob2)
        wait1(r0, ob)
        compute(r0, ob)

        @pl.when(r0 + 2 < EW_ROWS)
        def _():
            issue(r0 + 2, ob)

        wait1(r1, ob2)
        compute(r1, ob2)
        return carry

    lax.fori_loop(0, EW_ROWS // 2, pair, 0)
    pltpu.make_async_copy(norm3.at[wid], msgv, sem2).wait()
    plsc.subcore_barrier()
    pltpu.sync_copy(acc.at[pl.ds(off, SL)], pout.at[cid, pl.ds(off, SL)])


def _sc_iter(first, src3, dst3, norm3, fill_p, dinv_p, alpha16, pprev, opp):
    kern = pl.kernel(
        functools.partial(_sc_iter_body, first),
        out_type=[
            jax.ShapeDtypeStruct((NC, NP), jnp.float32),
            jax.ShapeDtypeStruct((NP,), jnp.float32),
        ],
        mesh=_mesh(),
        scratch_types=[
            pltpu.VMEM((SL,), jnp.float32),
            pltpu.VMEM((SL,), jnp.float32),
            pltpu.VMEM((SL,), jnp.float32),
            pltpu.VMEM((SL,), jnp.float32),
            pltpu.VMEM((SL,), jnp.float32),
            pltpu.VMEM((SL,), jnp.float32),
            pltpu.VMEM((16,), jnp.float32),
            pltpu.VMEM((SL,), jnp.float32),
            pltpu.VMEM((128,), jnp.float32),
            pltpu.VMEM((128,), jnp.float32),
            pltpu.VMEM((EW_ROWS, 128), jnp.int32),
            pltpu.VMEM((EW_ROWS, 128), jnp.int32),
            pltpu.VMEM((EW_ROWS, 128), jnp.float32),
            pltpu.VMEM((EW_ROWS, 128), jnp.float32),
            pltpu.VMEM_SHARED((NP,), jnp.float32),
            pltpu.VMEM_SHARED((NP,), jnp.float32),
            pltpu.SemaphoreType.DMA,
            pltpu.SemaphoreType.DMA,
        ],
    )
    return kern(src3, dst3, norm3, fill_p, dinv_p, alpha16, pprev, opp)


def kernel(x, mask, edge_index, W, b, alpha):
    src = edge_index[0]
    dst = edge_index[1]
    src3 = jnp.pad(src, (0, EP - E)).reshape(NW, EW_ROWS, 128)
    dst3 = jnp.pad(dst, (0, EP - E)).reshape(NW, EW_ROWS, 128)
    b2 = b.reshape(1, D)
    x_pad = jnp.pad(x, ((0, NP - N), (0, 0)))
    mask2 = jnp.pad(mask[:, 0], (0, NP - N)).reshape(EW_ROWS, 128)
    alpha16 = jnp.broadcast_to(alpha, (16,))
    alpha2 = jnp.broadcast_to(alpha, (1, 128))

    hn, fill2 = _tc_hn(x_pad, W, b2, mask2)
    fill_p = fill2.reshape(NP)
    ew3, degp = _sc_edges(hn, src3, dst3)
    dinv2 = _tc_dinv(degp.reshape(NC, EW_ROWS, 128))
    dinv_p = dinv2.reshape(NP)

    pprev = jnp.zeros((NC, NP), jnp.float32)
    opp = fill_p
    for t in range(K):
        pprev, opp = _sc_iter(t == 0, src3, dst3, ew3, fill_p, dinv_p,
                              alpha16, pprev, opp)

    out5 = _tc_blend(pprev.reshape(NC, EW_ROWS, 128),
                     opp.reshape(EW_ROWS, 128),
                     dinv_p.reshape(EW_ROWS, 128),
                     fill2, alpha2)
    out = out5.reshape(NP)[:N].reshape(N, 1)
    ew = ew3.reshape(EP)[:E]
    return (out, ew)
